# Initial kernel scaffold; baseline (speedup 1.0000x reference)
import jax
import jax.numpy as jnp
from jax.experimental import pallas as pl


def kernel(pointcloud, params):
    raise NotImplementedError("write your pallas kernel here")


# speedup vs baseline: 17.1241x; 17.1241x over previous
import functools

import jax
import jax.numpy as jnp
from jax import lax
from jax.experimental import pallas as pl
from jax.experimental.pallas import tpu as pltpu
from jax.experimental.pallas import tpu_sc as plsc

EPS = 1e-5
F32 = jnp.float32
I32 = jnp.int32


def _fps_stage(x, y, z, npoint):
    B, N = x.shape
    iota = lax.broadcasted_iota(I32, (B, N), 1)
    io_s = lax.broadcasted_iota(I32, (B, npoint), 1)

    def extract(last):
        m = iota == last
        cx = jnp.sum(jnp.where(m, x, 0.0), axis=1, keepdims=True)
        cy = jnp.sum(jnp.where(m, y, 0.0), axis=1, keepdims=True)
        cz = jnp.sum(jnp.where(m, z, 0.0), axis=1, keepdims=True)
        return cx, cy, cz

    def body(i, st):
        dist, last, fid, sx, sy, sz = st
        cx, cy, cz = extract(last)
        rec = io_s == (i - 1)
        sx = jnp.where(rec, cx, sx)
        sy = jnp.where(rec, cy, sy)
        sz = jnp.where(rec, cz, sz)
        d = (x - cx) ** 2 + (y - cy) ** 2 + (z - cz) ** 2
        dist = jnp.minimum(dist, d)
        mx = jnp.max(dist, axis=1, keepdims=True)
        nxt = jnp.min(jnp.where(dist >= mx, iota, N), axis=1, keepdims=True)
        fid = jnp.where(io_s == i, nxt, fid)
        return dist, nxt.astype(I32), fid, sx, sy, sz

    dist0 = jnp.full((B, N), 1e10, F32)
    last0 = jnp.zeros((B, 1), I32)
    fid0 = jnp.zeros((B, npoint), I32)
    sx0 = jnp.zeros((B, npoint), F32)
    sy0 = jnp.zeros((B, npoint), F32)
    sz0 = jnp.zeros((B, npoint), F32)
    dist, last, fid, sx, sy, sz = lax.fori_loop(
        1, npoint, body, (dist0, last0, fid0, sx0, sy0, sz0))
    cx, cy, cz = extract(last)
    rec = io_s == (npoint - 1)
    sx = jnp.where(rec, cx, sx)
    sy = jnp.where(rec, cy, sy)
    sz = jnp.where(rec, cz, sz)
    return fid, sx, sy, sz


def _fps_kernel(pc_ref, x1_ref, x2_ref, x3_ref, f2_ref, f3_ref):
    x = pc_ref[:, 0, :]
    y = pc_ref[:, 1, :]
    z = pc_ref[:, 2, :]
    _, sx1, sy1, sz1 = _fps_stage(x, y, z, 1024)
    f2, sx2, sy2, sz2 = _fps_stage(sx1, sy1, sz1, 512)
    f3, sx3, sy3, sz3 = _fps_stage(sx2, sy2, sz2, 256)
    x1_ref[:, 0, :] = sx1
    x1_ref[:, 1, :] = sy1
    x1_ref[:, 2, :] = sz1
    x2_ref[:, 0, :] = sx2
    x2_ref[:, 1, :] = sy2
    x2_ref[:, 2, :] = sz2
    x3_ref[:, 0, :] = sx3
    x3_ref[:, 1, :] = sy3
    x3_ref[:, 2, :] = sz3
    f2_ref[...] = f2
    f3_ref[...] = f3


def _run_fps(pc_t):
    B = pc_t.shape[0]
    return pl.pallas_call(
        _fps_kernel,
        out_shape=(
            jax.ShapeDtypeStruct((B, 3, 1024), F32),
            jax.ShapeDtypeStruct((B, 3, 512), F32),
            jax.ShapeDtypeStruct((B, 3, 256), F32),
            jax.ShapeDtypeStruct((B, 512), I32),
            jax.ShapeDtypeStruct((B, 256), I32),
        ),
    )(pc_t)


def _ballquery_kernel(xyz_ref, new_ref, idx_ref, cw_ref, *, r2, K, N, S, ST):
    b = pl.program_id(0)
    xyz = xyz_ref[0]
    new = new_ref[0]
    NC = N // 128
    NQ = (NC + 7) // 8

    sq_n = jnp.sum(new * new, axis=0)[:, None]
    sq_x = jnp.sum(xyz * xyz, axis=0)[None, :]
    dotp = lax.dot_general(new, xyz, (((0,), (0,)), ((), ())),
                           preferred_element_type=F32)
    d = sq_n + sq_x - 2.0 * dotp
    mask = (d <= r2).astype(F32)

    i_n = lax.broadcasted_iota(I32, (N, NC), 0)
    j_n = lax.broadcasted_iota(I32, (N, NC), 1)
    E = ((i_n // 128) == j_n).astype(F32)
    cs = lax.dot_general(mask, E, (((1,), (0,)), ((), ())),
                         preferred_element_type=F32)
    i_c = lax.broadcasted_iota(I32, (NC, NC), 0)
    j_c = lax.broadcasted_iota(I32, (NC, NC), 1)
    Tinc = (i_c <= j_c).astype(F32)
    c32 = lax.dot_general(cs, Tinc, (((1,), (0,)), ((), ())),
                          preferred_element_type=F32)
    excl = c32 - cs
    cnt = c32[:, NC - 1:NC]

    i_t = lax.broadcasted_iota(I32, (128, 128), 0)
    j_t = lax.broadcasted_iota(I32, (128, 128), 1)
    T128 = (i_t <= j_t).astype(F32)
    if NC < NQ * 8:
        cw_ref[...] = jnp.zeros((ST, NQ * 8, 128), F32)
    for j in range(NC):
        mj = mask[:, j * 128:(j + 1) * 128]
        cw_ref[:, j, :] = lax.dot_general(
            mj, T128, (((1,), (0,)), ((), ())), preferred_element_type=F32)

    kio = lax.broadcasted_iota(I32, (ST, K), 1).astype(F32)
    jk_cols = []
    for k in range(K):
        jk_cols.append(jnp.sum((c32 <= float(k)).astype(F32), axis=1,
                               keepdims=True))
    jk = jnp.concatenate(jk_cols, axis=1).astype(I32)
    jk = jnp.minimum(jk, NC - 1)

    base = jnp.take_along_axis(excl, jk, axis=1,
                               mode="promise_in_bounds")
    tk = kio - base

    jq3 = jnp.broadcast_to((jk % 8)[:, :, None], (ST, K, 128))
    hi3 = jnp.broadcast_to((jk // 8)[:, :, None], (ST, K, 128))
    rows = jnp.zeros((ST, K, 128), F32)
    for q in range(NQ):
        slab = cw_ref[:, q * 8:(q + 1) * 8, :]
        g = jnp.take_along_axis(slab, jq3, axis=1, mode="promise_in_bounds")
        rows = jnp.where(hi3 == q, g, rows)
    pos = jnp.sum((rows <= tk[:, :, None]).astype(F32), axis=2)

    lidx = jk * 128 + pos.astype(I32)
    valid = kio < cnt
    first = jnp.broadcast_to(lidx[:, 0:1], (ST, K))
    nonempty = cnt > 0.0
    lidx = jnp.where(valid, lidx, jnp.where(nonempty, first, 0))
    idx_ref[0] = lidx + b * N


def _run_ballquery(xyz_t, new_t, radius, K):
    B, _, N = xyz_t.shape
    S = new_t.shape[2]
    ST = min(S, 256)
    NC = N // 128
    NQ = (NC + 7) // 8
    kern = functools.partial(_ballquery_kernel, r2=float(radius) ** 2, K=K,
                             N=N, S=S, ST=ST)
    return pl.pallas_call(
        kern,
        grid=(B, S // ST),
        in_specs=[
            pl.BlockSpec((1, 3, N), lambda b, t: (b, 0, 0)),
            pl.BlockSpec((1, 3, ST), lambda b, t: (b, 0, t)),
        ],
        out_specs=pl.BlockSpec((1, ST, K), lambda b, t: (b, t, 0)),
        out_shape=jax.ShapeDtypeStruct((B, S, K), I32),
        scratch_shapes=[pltpu.VMEM((ST, NQ * 8, 128), F32)],
    )(xyz_t, new_t)


def _sc_gather(table, idx):
    M = idx.shape[0]
    D = table.shape[1]
    info = plsc.get_sparse_core_info()
    NW = info.num_cores * info.num_subcores
    per_w = M // NW
    CH = 128
    steps = per_w // CH
    mesh = plsc.VectorSubcoreMesh(core_axis_name="c", subcore_axis_name="s")

    @functools.partial(
        pl.kernel, mesh=mesh,
        out_type=jax.ShapeDtypeStruct((M, D), F32),
        scratch_types=[
            pltpu.VMEM((CH,), I32),
            pltpu.VMEM((CH, D), F32),
            pltpu.SemaphoreType.DMA,
        ],
    )
    def k(table_hbm, idx_hbm, out_hbm, idx_v, rows_v, sem):
        wid = lax.axis_index("s") * info.num_cores + lax.axis_index("c")
        base = wid * per_w

        def body(t, carry):
            off = base + t * CH
            pltpu.sync_copy(idx_hbm.at[pl.ds(off, CH)], idx_v)
            pltpu.async_copy(table_hbm.at[idx_v], rows_v, sem).wait()
            pltpu.sync_copy(rows_v, out_hbm.at[pl.ds(off, CH)])
            return carry

        lax.fori_loop(0, steps, body, 0)

    return k(table, idx)


def _stats_kernel(rows_ref, ctr_ref, w_ref, s1_ref, s2_ref, *, ST, K, Cp, O):
    t = pl.program_id(0) * pl.num_programs(1) + pl.program_id(1)

    @pl.when(t == 0)
    def _():
        s1_ref[...] = jnp.zeros_like(s1_ref)
        s2_ref[...] = jnp.zeros_like(s2_ref)

    g = rows_ref[0].reshape(ST, K, Cp) - ctr_ref[0][:, None, :]
    y = lax.dot_general(g.reshape(ST * K, Cp), w_ref[...],
                        (((1,), (1,)), ((), ())),
                        preferred_element_type=F32)
    s1_ref[...] += jnp.sum(y, axis=0)[None, :]
    s2_ref[...] += jnp.sum(y * y, axis=0)[None, :]


def _run_stats(rows, ctr, W, ST):
    B, SK, Cp = rows.shape
    S = ctr.shape[1]
    K = SK // S
    O = W.shape[0]
    kern = functools.partial(_stats_kernel, ST=ST, K=K, Cp=Cp, O=O)
    return pl.pallas_call(
        kern,
        grid=(B, S // ST),
        in_specs=[
            pl.BlockSpec((1, ST * K, Cp), lambda b, t: (b, t, 0)),
            pl.BlockSpec((1, ST, Cp), lambda b, t: (b, t, 0)),
            pl.BlockSpec(W.shape, lambda b, t: (0, 0)),
        ],
        out_specs=[
            pl.BlockSpec((1, O), lambda b, t: (0, 0)),
            pl.BlockSpec((1, O), lambda b, t: (0, 0)),
        ],
        out_shape=[
            jax.ShapeDtypeStruct((1, O), F32),
            jax.ShapeDtypeStruct((1, O), F32),
        ],
    )(rows, ctr, W)


def _layer2_kernel(rows_ref, ctr_ref, w_ref, s1_ref, s2_ref, w2_ref,
                   p_ref, t1_ref, t2_ref, *, ST, K, Cp, O, O2, count):
    t = pl.program_id(0) * pl.num_programs(1) + pl.program_id(1)

    @pl.when(t == 0)
    def _():
        t1_ref[...] = jnp.zeros_like(t1_ref)
        t2_ref[...] = jnp.zeros_like(t2_ref)

    m = s1_ref[...] / count
    v = s2_ref[...] / count - m * m
    g = rows_ref[0].reshape(ST, K, Cp) - ctr_ref[0][:, None, :]
    y1 = lax.dot_general(g.reshape(ST * K, Cp), w_ref[...],
                         (((1,), (1,)), ((), ())),
                         preferred_element_type=F32)
    h1 = jnp.maximum((y1 - m) / jnp.sqrt(v + EPS), 0.0)
    y2 = lax.dot_general(h1, w2_ref[...], (((1,), (1,)), ((), ())),
                         preferred_element_type=F32)
    t1_ref[...] += jnp.sum(y2, axis=0)[None, :]
    t2_ref[...] += jnp.sum(y2 * y2, axis=0)[None, :]
    p_ref[0] = jnp.max(y2.reshape(ST, K, O2), axis=1)


def _run_layer2(rows, ctr, W, s1, s2, W2, count, ST):
    B, SK, Cp = rows.shape
    S = ctr.shape[1]
    K = SK // S
    O = W.shape[0]
    O2 = W2.shape[0]
    kern = functools.partial(_layer2_kernel, ST=ST, K=K, Cp=Cp, O=O, O2=O2,
                             count=float(count))
    return pl.pallas_call(
        kern,
        grid=(B, S // ST),
        in_specs=[
            pl.BlockSpec((1, ST * K, Cp), lambda b, t: (b, t, 0)),
            pl.BlockSpec((1, ST, Cp), lambda b, t: (b, t, 0)),
            pl.BlockSpec(W.shape, lambda b, t: (0, 0)),
            pl.BlockSpec((1, O), lambda b, t: (0, 0)),
            pl.BlockSpec((1, O), lambda b, t: (0, 0)),
            pl.BlockSpec(W2.shape, lambda b, t: (0, 0)),
        ],
        out_specs=[
            pl.BlockSpec((1, ST, O2), lambda b, t: (b, t, 0)),
            pl.BlockSpec((1, O2), lambda b, t: (0, 0)),
            pl.BlockSpec((1, O2), lambda b, t: (0, 0)),
        ],
        out_shape=[
            jax.ShapeDtypeStruct((B, S, O2), F32),
            jax.ShapeDtypeStruct((1, O2), F32),
            jax.ShapeDtypeStruct((1, O2), F32),
        ],
    )(rows, ctr, W, s1, s2, W2)


def _pool_kernel(rows_ref, ctr_ref, w_ref, p_ref, s1_ref, s2_ref,
                 *, ST, K, Cp, O):
    t = pl.program_id(0) * pl.num_programs(1) + pl.program_id(1)

    @pl.when(t == 0)
    def _():
        s1_ref[...] = jnp.zeros_like(s1_ref)
        s2_ref[...] = jnp.zeros_like(s2_ref)

    g = rows_ref[0].reshape(ST, K, Cp) - ctr_ref[0][:, None, :]
    y = lax.dot_general(g.reshape(ST * K, Cp), w_ref[...],
                        (((1,), (1,)), ((), ())),
                        preferred_element_type=F32)
    s1_ref[...] += jnp.sum(y, axis=0)[None, :]
    s2_ref[...] += jnp.sum(y * y, axis=0)[None, :]
    p_ref[0] = jnp.max(y.reshape(ST, K, O), axis=1)


def _run_pool(rows, ctr, W, ST):
    B, SK, Cp = rows.shape
    S = ctr.shape[1]
    K = SK // S
    O = W.shape[0]
    kern = functools.partial(_pool_kernel, ST=ST, K=K, Cp=Cp, O=O)
    return pl.pallas_call(
        kern,
        grid=(B, S // ST),
        in_specs=[
            pl.BlockSpec((1, ST * K, Cp), lambda b, t: (b, t, 0)),
            pl.BlockSpec((1, ST, Cp), lambda b, t: (b, t, 0)),
            pl.BlockSpec(W.shape, lambda b, t: (0, 0)),
        ],
        out_specs=[
            pl.BlockSpec((1, ST, O), lambda b, t: (b, t, 0)),
            pl.BlockSpec((1, O), lambda b, t: (0, 0)),
            pl.BlockSpec((1, O), lambda b, t: (0, 0)),
        ],
        out_shape=[
            jax.ShapeDtypeStruct((B, S, O), F32),
            jax.ShapeDtypeStruct((1, O), F32),
            jax.ShapeDtypeStruct((1, O), F32),
        ],
    )(rows, ctr, W)


def _finalize_kernel(p_ref, s1_ref, s2_ref, f_ref, *, count):
    p = p_ref[0]
    m = s1_ref[...] / count
    v = s2_ref[...] / count - m * m
    f_ref[0] = jnp.maximum((p - m) / jnp.sqrt(v + EPS), 0.0)


def _run_finalize(p, s1, s2, count):
    B, S, O = p.shape
    kern = functools.partial(_finalize_kernel, count=float(count))
    return pl.pallas_call(
        kern,
        grid=(B,),
        in_specs=[
            pl.BlockSpec((1, S, O), lambda b: (b, 0, 0)),
            pl.BlockSpec((1, O), lambda b: (0, 0)),
            pl.BlockSpec((1, O), lambda b: (0, 0)),
        ],
        out_specs=pl.BlockSpec((1, S, O), lambda b: (b, 0, 0)),
        out_shape=jax.ShapeDtypeStruct((B, S, O), F32),
    )(p, s1, s2)


def _pre_kernel(d0_ref, d1_ref, wd0_ref, wd1_ref,
                r0_ref, r1_ref, a1_ref, a2_ref, b1_ref, b2_ref):
    t = pl.program_id(0)

    @pl.when(t == 0)
    def _():
        a1_ref[...] = jnp.zeros_like(a1_ref)
        a2_ref[...] = jnp.zeros_like(a2_ref)
        b1_ref[...] = jnp.zeros_like(b1_ref)
        b2_ref[...] = jnp.zeros_like(b2_ref)

    r0 = lax.dot_general(d0_ref[0], wd0_ref[...], (((1,), (1,)), ((), ())),
                         preferred_element_type=F32)
    r1 = lax.dot_general(d1_ref[0], wd1_ref[...], (((1,), (1,)), ((), ())),
                         preferred_element_type=F32)
    r0_ref[0] = r0
    r1_ref[0] = r1
    a1_ref[...] += jnp.sum(r0, axis=0)[None, :]
    a2_ref[...] += jnp.sum(r0 * r0, axis=0)[None, :]
    b1_ref[...] += jnp.sum(r1, axis=0)[None, :]
    b2_ref[...] += jnp.sum(r1 * r1, axis=0)[None, :]


def _y4_kernel(x3_ref, r0_ref, r1_ref, p3_ref, a1_ref, a2_ref, b1_ref,
               b2_ref, s31_ref, s32_ref, w4_ref, pool_ref, s41_ref, s42_ref,
               *, S, cds, c3):
    t = pl.program_id(0)

    @pl.when(t == 0)
    def _():
        s41_ref[...] = jnp.zeros_like(s41_ref)
        s42_ref[...] = jnp.zeros_like(s42_ref)

    def norm(x, s1, s2, cnt):
        m = s1 / cnt
        v = s2 / cnt - m * m
        return jnp.maximum((x - m) / jnp.sqrt(v + EPS), 0.0)

    r0 = norm(r0_ref[0], a1_ref[...], a2_ref[...], cds)
    r1 = norm(r1_ref[0], b1_ref[...], b2_ref[...], cds)
    f3 = norm(p3_ref[0], s31_ref[...], s32_ref[...], c3)
    xyz_rows = jnp.transpose(x3_ref[0], (1, 0))
    grouped = jnp.concatenate([xyz_rows, r0, r1, f3], axis=1)
    y4 = lax.dot_general(grouped, w4_ref[...], (((1,), (1,)), ((), ())),
                         preferred_element_type=F32)
    s41_ref[...] += jnp.sum(y4, axis=0)[None, :]
    s42_ref[...] += jnp.sum(y4 * y4, axis=0)[None, :]
    pool_ref[0] = jnp.max(y4, axis=0, keepdims=True)


def _fc_kernel(pool_ref, s41_ref, s42_ref, wf1_ref, wf2_ref, out_ref, *, c4):
    m4 = s41_ref[...] / c4
    v4 = s42_ref[...] / c4 - m4 * m4
    g = jnp.maximum((pool_ref[...] - m4) / jnp.sqrt(v4 + EPS), 0.0)

    def bnrelu(x):
        cnt = x.shape[0]
        m = jnp.sum(x, axis=0, keepdims=True) / cnt
        v = jnp.sum(x * x, axis=0, keepdims=True) / cnt - m * m
        return jnp.maximum((x - m) / jnp.sqrt(v + EPS), 0.0)

    h = bnrelu(lax.dot_general(g, wf1_ref[...], (((1,), (1,)), ((), ())),
                               preferred_element_type=F32))
    out_ref[...] = bnrelu(lax.dot_general(h, wf2_ref[...],
                                          (((1,), (1,)), ((), ())),
                                          preferred_element_type=F32))


def _run_final(x3_t, d0rows, d1rows, p3, s31, s32, Wd0, Wd1, W4, Wf1, Wf2):
    B, _, S = x3_t.shape
    c3 = float(B * S * 64)
    cds = float(B * S)
    d0 = d0rows.reshape(B, S, 128)
    d1 = d1rows.reshape(B, S, 256)
    r0p, r1p, a1, a2, b1, b2 = pl.pallas_call(
        _pre_kernel,
        grid=(B,),
        in_specs=[
            pl.BlockSpec((1, S, 128), lambda b: (b, 0, 0)),
            pl.BlockSpec((1, S, 256), lambda b: (b, 0, 0)),
            pl.BlockSpec(Wd0.shape, lambda b: (0, 0)),
            pl.BlockSpec(Wd1.shape, lambda b: (0, 0)),
        ],
        out_specs=[
            pl.BlockSpec((1, S, 256), lambda b: (b, 0, 0)),
            pl.BlockSpec((1, S, 256), lambda b: (b, 0, 0)),
            pl.BlockSpec((1, 256), lambda b: (0, 0)),
            pl.BlockSpec((1, 256), lambda b: (0, 0)),
            pl.BlockSpec((1, 256), lambda b: (0, 0)),
            pl.BlockSpec((1, 256), lambda b: (0, 0)),
        ],
        out_shape=[
            jax.ShapeDtypeStruct((B, S, 256), F32),
            jax.ShapeDtypeStruct((B, S, 256), F32),
            jax.ShapeDtypeStruct((1, 256), F32),
            jax.ShapeDtypeStruct((1, 256), F32),
            jax.ShapeDtypeStruct((1, 256), F32),
            jax.ShapeDtypeStruct((1, 256), F32),
        ],
    )(d0, d1, Wd0, Wd1)

    kern4 = functools.partial(_y4_kernel, S=S, cds=cds, c3=c3)
    pooled, s41, s42 = pl.pallas_call(
        kern4,
        grid=(B,),
        in_specs=[
            pl.BlockSpec((1, 3, S), lambda b: (b, 0, 0)),
            pl.BlockSpec((1, S, 256), lambda b: (b, 0, 0)),
            pl.BlockSpec((1, S, 256), lambda b: (b, 0, 0)),
            pl.BlockSpec((1, S, 256), lambda b: (b, 0, 0)),
            pl.BlockSpec((1, 256), lambda b: (0, 0)),
            pl.BlockSpec((1, 256), lambda b: (0, 0)),
            pl.BlockSpec((1, 256), lambda b: (0, 0)),
            pl.BlockSpec((1, 256), lambda b: (0, 0)),
            pl.BlockSpec((1, 256), lambda b: (0, 0)),
            pl.BlockSpec((1, 256), lambda b: (0, 0)),
            pl.BlockSpec(W4.shape, lambda b: (0, 0)),
        ],
        out_specs=[
            pl.BlockSpec((1, 1, 1024), lambda b: (b, 0, 0)),
            pl.BlockSpec((1, 1024), lambda b: (0, 0)),
            pl.BlockSpec((1, 1024), lambda b: (0, 0)),
        ],
        out_shape=[
            jax.ShapeDtypeStruct((B, 1, 1024), F32),
            jax.ShapeDtypeStruct((1, 1024), F32),
            jax.ShapeDtypeStruct((1, 1024), F32),
        ],
    )(x3_t, r0p, r1p, p3, a1, a2, b1, b2, s31, s32, W4)
    pooled = pooled.reshape(B, 1024)

    kernf = functools.partial(_fc_kernel, c4=cds)
    return pl.pallas_call(
        kernf,
        out_shape=jax.ShapeDtypeStruct((B, 512), F32),
    )(pooled, s41, s42, Wf1, Wf2)


def kernel(pointcloud, params):
    B, N, _ = pointcloud.shape
    pc_t = jnp.transpose(pointcloud, (0, 2, 1))

    W1a, _, _ = params['sa1'][0]
    W1b, _, _ = params['sa1'][1]
    W2a, _, _ = params['sa2'][0]
    W2b, _, _ = params['sa2'][1]
    W3a, _, _ = params['sa3'][0]
    Wd0, _, _ = params['ds0']
    Wd1, _, _ = params['ds1']
    W4, _, _ = params['sa4'][0]
    Wf1, _, _ = params['fc'][0]
    Wf2, _, _ = params['fc'][1]

    def padc(x, c):
        return jnp.pad(x, ((0, 0),) * (x.ndim - 1) + ((0, c - x.shape[-1]),))

    x1_t, x2_t, x3_t, fid2, fid3 = _run_fps(pc_t)
    idx1 = _run_ballquery(pc_t, x1_t, 0.23, 48)
    idx2 = _run_ballquery(x1_t, x2_t, 0.32, 64)
    idx3 = _run_ballquery(x2_t, x3_t, 0.32, 64)

    x1r = jnp.transpose(x1_t, (0, 2, 1))
    x2r = jnp.transpose(x2_t, (0, 2, 1))
    x3r = jnp.transpose(x3_t, (0, 2, 1))

    S1, K1, O1 = 1024, 48, 128
    C1 = 128
    T1 = padc(pointcloud.reshape(B * N, 3), C1)
    rows1 = _sc_gather(T1, idx1.reshape(B * S1 * K1)).reshape(B, S1 * K1, C1)
    ctr1 = padc(x1r, C1)
    W1ap = padc(W1a, C1)
    c1 = B * S1 * K1
    s11, s12 = _run_stats(rows1, ctr1, W1ap, 128)
    p2, t21, t22 = _run_layer2(rows1, ctr1, W1ap, s11, s12, W1b, c1, 128)
    f1 = _run_finalize(p2, t21, t22, c1)

    S2, K2, O2 = 512, 64, 256
    C2 = 256
    T2 = padc(jnp.concatenate([x1r, f1], axis=2).reshape(B * S1, 131), C2)
    rows2 = _sc_gather(T2, idx2.reshape(B * S2 * K2)).reshape(B, S2 * K2, C2)
    ctr2 = padc(x2r, C2)
    W2ap = padc(W2a, C2)
    c2 = B * S2 * K2
    s21, s22 = _run_stats(rows2, ctr2, W2ap, 64)
    p3q, t31, t32 = _run_layer2(rows2, ctr2, W2ap, s21, s22, W2b, c2, 64)
    f2 = _run_finalize(p3q, t31, t32, c2)

    S3, K3, O3 = 256, 64, 256
    C3 = 384
    T3 = padc(jnp.concatenate([x2r, f2], axis=2).reshape(B * S2, 259), C3)
    rows3 = _sc_gather(T3, idx3.reshape(B * S3 * K3)).reshape(B, S3 * K3, C3)
    ctr3 = padc(x3r, C3)
    W3ap = padc(W3a, C3)
    p3, s31, s32 = _run_pool(rows3, ctr3, W3ap, 64)

    boff1 = (jnp.arange(B, dtype=I32) * S1)[:, None]
    boff2 = (jnp.arange(B, dtype=I32) * S2)[:, None]
    gd0 = (fid2[:, :256] + boff1).reshape(B * 256)
    gd1 = (fid3 + boff2).reshape(B * 256)
    d0rows = _sc_gather(T2, gd0)[:, 3:131]
    d1rows = _sc_gather(T3, gd1)[:, 3:259]

    return _run_final(x3_t, d0rows, d1rows, p3, s31, s32,
                      Wd0, Wd1, W4, Wf1, Wf2)
